# baseline (speedup 1.0000x reference)
import functools

import jax
import jax.numpy as jnp
from jax import lax
from jax.experimental import pallas as pl
from jax.experimental.pallas import tpu as pltpu
from jax.experimental.pallas import tpu_sc as plsc

B = 10000
NN = 100000
F = 128
DO = 256
K = 32

NC = 2
NS = 16
NW = NC * NS
BP = 10240
RPW = BP // NW

_MESH = dict(
    mesh=plsc.VectorSubcoreMesh(core_axis_name="c", subcore_axis_name="s"),
    compiler_params=pltpu.CompilerParams(needs_layout_passes=False),
)


def _wid():
    return lax.axis_index("s") * NC + lax.axis_index("c")


NWIN_SC = B // 16
NWIN_GA = BP // 16


@functools.partial(
    pl.kernel,
    out_type=jax.ShapeDtypeStruct((BP,), jnp.int32),
    scratch_types=[
        pltpu.VMEM((BP,), jnp.int32),
        pltpu.VMEM((NN,), jnp.int32),
        pltpu.VMEM((BP,), jnp.int32),
    ],
    **_MESH,
)
def _fmap(nodes_hbm, f_hbm, nodes_v, tab_v, f_v):
    @pl.when(_wid() == 0)
    def _():
        pltpu.sync_copy(nodes_hbm, nodes_v)
        iota = lax.iota(jnp.int32, 16)

        def scatter_win(i, carry):
            w = (NWIN_SC - 1) - i
            base = w * 16
            vn = nodes_v[pl.ds(base, 16)]
            dup = iota < 0
            for k in range(1, 16):
                idx_k = base + iota - k
                valid = jnp.logical_or(idx_k >= 0, iota >= 1)
                prev = plsc.load_gather(nodes_v, [jnp.maximum(idx_k, 0)])
                dup = jnp.logical_or(
                    dup, jnp.logical_and(prev == vn, valid)
                )
            keep = jnp.logical_not(dup)
            plsc.store_scatter(tab_v, [vn], base + iota, mask=keep)
            return carry

        lax.fori_loop(0, NWIN_SC, scatter_win, 0)

        def gather_win(w, carry):
            base = w * 16
            vn = nodes_v[pl.ds(base, 16)]
            fv = plsc.load_gather(tab_v, [vn])
            fv = jnp.minimum(jnp.maximum(fv, 0), B - 1)
            f_v[pl.ds(base, 16)] = fv
            return carry

        lax.fori_loop(0, NWIN_GA, gather_win, 0)
        pltpu.sync_copy(f_v, f_hbm)


RC = 4
NCH = RPW // RC


NBUF = 4
C0 = 40
C1 = 120
CT = C0 + C1
CMAX = max(C0, C1)


@functools.partial(
    pl.kernel,
    out_type=jax.ShapeDtypeStruct((BP, F), jnp.float32),
    scratch_types=[
        pltpu.VMEM((CMAX, RC * K), jnp.int32),
        pltpu.VMEM((NBUF, RC * K, F), jnp.float32),
        pltpu.VMEM((NBUF, RC, F), jnp.float32),
        [pltpu.SemaphoreType.DMA] * NBUF,
        [pltpu.SemaphoreType.DMA] * NBUF,
    ],
    **_MESH,
)
def _gsum(nf2_hbm, feats_hbm, s_hbm, idx_v, buf_v, oring_v, gsems, osems):
    c_ax = lax.axis_index("c")
    s_ax = lax.axis_index("s")
    cb = s_ax * CT + c_ax * C0
    nch_w = jnp.where(c_ax == 0, C0, C1)
    pltpu.sync_copy(nf2_hbm.at[pl.ds(cb, CMAX)], idx_v)
    for p in range(NBUF - 1):
        pltpu.async_copy(feats_hbm.at[idx_v.at[p]], buf_v.at[p], gsems[p])

    def outer(i, carry):
        c0 = i * NBUF
        for b in range(NBUF):
            c = c0 + b
            nxt = c + NBUF - 1
            bn = (b + NBUF - 1) % NBUF

            @pl.when(nxt < nch_w)
            def _():
                pltpu.async_copy(
                    feats_hbm.at[idx_v.at[nxt]], buf_v.at[bn], gsems[bn]
                )

            pltpu.make_async_copy(
                feats_hbm.at[idx_v.at[c]], buf_v.at[b], gsems[b]
            ).wait()

            @pl.when(c >= NBUF)
            def _():
                pltpu.make_async_copy(
                    oring_v.at[b], s_hbm.at[pl.ds(0, RC)], osems[b]
                ).wait()

            def row_body(r, rcarry):
                for v in range(F // 16):
                    sl = pl.ds(v * 16, 16)
                    acc = buf_v[b, r * K, sl]
                    for k in range(1, K):
                        acc = acc + buf_v[b, r * K + k, sl]
                    oring_v[b, r, sl] = acc
                return rcarry

            lax.fori_loop(0, RC, row_body, 0)
            pltpu.async_copy(
                oring_v.at[b], s_hbm.at[pl.ds((cb + c) * RC, RC)], osems[b]
            )
        return carry

    lax.fori_loop(0, nch_w // NBUF, outer, 0)
    for b in range(NBUF):
        pltpu.make_async_copy(
            oring_v.at[b], s_hbm.at[pl.ds(0, RC)], osems[b]
        ).wait()


BM = 1024


def _mm_body(x_ref, s_ref, w1_ref, w2_ref, b_ref, o_ref):
    acc = jnp.dot(x_ref[...], w1_ref[...], preferred_element_type=jnp.float32)
    acc = acc + jnp.dot(
        s_ref[...] * (1.0 / K), w2_ref[...], preferred_element_type=jnp.float32
    )
    o_ref[...] = jnp.maximum(acc + b_ref[...], 0.0)


def _matmul(x_p, s, w1, w2, b2):
    return pl.pallas_call(
        _mm_body,
        grid=(BP // BM,),
        in_specs=[
            pl.BlockSpec((BM, F), lambda i: (i, 0)),
            pl.BlockSpec((BM, F), lambda i: (i, 0)),
            pl.BlockSpec((F, DO), lambda i: (0, 0)),
            pl.BlockSpec((F, DO), lambda i: (0, 0)),
            pl.BlockSpec((1, DO), lambda i: (0, 0)),
        ],
        out_specs=pl.BlockSpec((BM, DO), lambda i: (i, 0)),
        out_shape=jax.ShapeDtypeStruct((BP, DO), jnp.float32),
    )(x_p, s, w1, w2, b2)


RCO = 80
NCO = RPW // RCO


@functools.partial(
    pl.kernel,
    out_type=jax.ShapeDtypeStruct((BP, DO), jnp.float32),
    scratch_types=[
        pltpu.VMEM((NCO, RCO), jnp.int32),
        pltpu.VMEM((2, RCO, DO), jnp.float32),
        [pltpu.SemaphoreType.DMA] * 2,
    ],
    **_MESH,
)
def _gout(g_hbm, f2_hbm, o_hbm, idx_v, buf_v, sems):
    w = _wid()
    pltpu.sync_copy(f2_hbm.at[pl.ds(w * NCO, NCO)], idx_v)
    pltpu.async_copy(g_hbm.at[idx_v.at[0]], buf_v.at[0], sems[0])

    def outer(i, carry):
        c0 = i * 2
        for b in range(2):
            c = c0 + b

            @pl.when(c + 1 < NCO)
            def _():
                pltpu.async_copy(
                    g_hbm.at[idx_v.at[c + 1]], buf_v.at[1 - b], sems[1 - b]
                )

            pltpu.make_async_copy(
                g_hbm.at[idx_v.at[c]], buf_v.at[b], sems[b]
            ).wait()
            pltpu.sync_copy(buf_v.at[b], o_hbm.at[pl.ds(w * RPW + c * RCO, RCO)])
        return carry

    lax.fori_loop(0, NCO // 2, outer, 0)


def kernel(x, nodes, neighs, feats, W, b):
    nodes_p = jnp.concatenate(
        [nodes.astype(jnp.int32), jnp.zeros((BP - B,), jnp.int32)]
    )
    neighs_p = jnp.concatenate(
        [neighs.astype(jnp.int32), jnp.zeros((BP - B, K), jnp.int32)]
    )
    nf = jnp.concatenate(
        [
            jnp.reshape(neighs_p, (BP * K // 128, 128)),
            jnp.zeros((CMAX, 128), jnp.int32),
        ]
    )
    x_p = jnp.concatenate([x, jnp.zeros((BP - B, F), jnp.float32)])
    w1 = W[:F]
    w2 = W[F:]
    b2 = jnp.reshape(b, (1, DO))

    f = _fmap(nodes_p)
    f2 = jnp.reshape(f, (BP // RCO, RCO))
    s = _gsum(nf, feats)
    g = _matmul(x_p, s, w1, w2, b2)
    out = _gout(g, f2)
    return out[:B]


# speedup vs baseline: 1.0951x; 1.0951x over previous
import functools

import jax
import jax.numpy as jnp
from jax import lax
from jax.experimental import pallas as pl
from jax.experimental.pallas import tpu as pltpu
from jax.experimental.pallas import tpu_sc as plsc

B = 10000
NN = 100000
F = 128
DO = 256
K = 32

NC = 2
NS = 16
NW = NC * NS
BP = 10240
RPW = BP // NW

_MESH = dict(
    mesh=plsc.VectorSubcoreMesh(core_axis_name="c", subcore_axis_name="s"),
    compiler_params=pltpu.CompilerParams(needs_layout_passes=False),
)


def _wid():
    return lax.axis_index("s") * NC + lax.axis_index("c")


def _vgather16(vec, idx):
    dnums = lax.GatherDimensionNumbers(
        offset_dims=(), collapsed_slice_dims=(0,), start_index_map=(0,)
    )
    return lax.gather(
        vec, idx[:, None], dnums, (1,),
        mode=lax.GatherScatterMode.PROMISE_IN_BOUNDS,
    )


NWIN_SC = B // 16
NWIN_GA = BP // 16


@functools.partial(
    pl.kernel,
    out_type=jax.ShapeDtypeStruct((BP,), jnp.int32),
    scratch_types=[
        pltpu.VMEM((BP,), jnp.int32),
        pltpu.VMEM((NN,), jnp.int32),
        pltpu.VMEM((BP,), jnp.int32),
        pltpu.VMEM((16,), jnp.int32),
    ],
    **_MESH,
)
def _fmap(nodes_hbm, f_hbm, nodes_v, tab_v, wf_v, win_v):
    @pl.when(_wid() == 0)
    def _():
        pltpu.sync_copy(nodes_hbm, nodes_v)
        iota = lax.iota(jnp.int32, 16)

        def p1(w, carry):
            base = w * 16
            vn = nodes_v[pl.ds(base, 16)]
            plsc.store_scatter(tab_v, [vn], base + iota)
            return carry

        lax.fori_loop(0, NWIN_SC, p1, 0)

        def p2(w, off):
            base = w * 16
            vn = nodes_v[pl.ds(base, 16)]
            f0 = plsc.load_gather(tab_v, [vn])
            lose = f0 != base + iota
            cs = plsc.cumsum(lose.astype(jnp.int32))
            pos = off + cs - 1
            plsc.store_scatter(wf_v, [pos], base + iota, mask=lose)
            return off + cs[15]

        nwl = lax.fori_loop(0, NWIN_SC, p2, 0)

        def p3(w, carry):
            base = w * 16
            wi = wf_v[pl.ds(base, 16)]
            validm = (base + iota) < nwl
            wi = jnp.minimum(jnp.maximum(wi, 0), B - 1)
            vn = plsc.load_gather(nodes_v, [wi])
            dup = iota < 0
            for k in range(1, 16):
                prev = _vgather16(vn, jnp.maximum(iota - k, 0))
                dup = jnp.logical_or(
                    dup, jnp.logical_and(prev == vn, iota >= k)
                )
            keep = jnp.logical_and(jnp.logical_not(dup), validm)
            cur = plsc.load_gather(tab_v, [vn])
            plsc.store_scatter(tab_v, [vn], jnp.minimum(cur, wi), mask=keep)
            return carry

        lax.fori_loop(0, (nwl + 15) // 16, p3, 0)

        def p4(w, carry):
            base = w * 16
            vn = nodes_v[pl.ds(base, 16)]
            fv = plsc.load_gather(tab_v, [vn])
            fv = jnp.minimum(jnp.maximum(fv, 0), B - 1)
            wf_v[pl.ds(base, 16)] = fv
            return carry

        lax.fori_loop(0, NWIN_GA, p4, 0)
        pltpu.sync_copy(wf_v, f_hbm)


RC = 4
NCH = RPW // RC


NBUF = 4


@functools.partial(
    pl.kernel,
    out_type=jax.ShapeDtypeStruct((BP, F), jnp.float32),
    scratch_types=[
        pltpu.VMEM((NCH, RC * K), jnp.int32),
        pltpu.VMEM((NBUF, RC * K, F), jnp.float32),
        pltpu.VMEM((RPW, F), jnp.float32),
        [pltpu.SemaphoreType.DMA] * NBUF,
    ],
    **_MESH,
)
def _gsum(nf2_hbm, feats_hbm, s_hbm, idx_v, buf_v, acc_v, sems):
    w = _wid()
    pltpu.sync_copy(nf2_hbm.at[pl.ds(w * NCH, NCH)], idx_v)
    for p in range(NBUF - 1):
        pltpu.async_copy(feats_hbm.at[idx_v.at[p]], buf_v.at[p], sems[p])

    def outer(i, carry):
        c0 = i * NBUF
        for b in range(NBUF):
            c = c0 + b
            nxt = c + NBUF - 1
            bn = (b + NBUF - 1) % NBUF

            @pl.when(nxt < NCH)
            def _():
                pltpu.async_copy(
                    feats_hbm.at[idx_v.at[nxt]], buf_v.at[bn], sems[bn]
                )

            pltpu.make_async_copy(
                feats_hbm.at[idx_v.at[c]], buf_v.at[b], sems[b]
            ).wait()

            def row_body(r, rcarry):
                for v in range(F // 16):
                    sl = pl.ds(v * 16, 16)
                    acc = buf_v[b, r * K, sl]
                    for k in range(1, K):
                        acc = acc + buf_v[b, r * K + k, sl]
                    acc_v[c * RC + r, sl] = acc
                return rcarry

            lax.fori_loop(0, RC, row_body, 0)
        return carry

    lax.fori_loop(0, NCH // NBUF, outer, 0)
    pltpu.sync_copy(acc_v, s_hbm.at[pl.ds(w * RPW, RPW)])


BM = 1024


def _mm_body(x_ref, s_ref, w1_ref, w2_ref, b_ref, o_ref):
    acc = jnp.dot(x_ref[...], w1_ref[...], preferred_element_type=jnp.float32)
    acc = acc + jnp.dot(
        s_ref[...] * (1.0 / K), w2_ref[...], preferred_element_type=jnp.float32
    )
    o_ref[...] = jnp.maximum(acc + b_ref[...], 0.0)


def _matmul(x_p, s, w1, w2, b2):
    return pl.pallas_call(
        _mm_body,
        grid=(BP // BM,),
        in_specs=[
            pl.BlockSpec((BM, F), lambda i: (i, 0)),
            pl.BlockSpec((BM, F), lambda i: (i, 0)),
            pl.BlockSpec((F, DO), lambda i: (0, 0)),
            pl.BlockSpec((F, DO), lambda i: (0, 0)),
            pl.BlockSpec((1, DO), lambda i: (0, 0)),
        ],
        out_specs=pl.BlockSpec((BM, DO), lambda i: (i, 0)),
        out_shape=jax.ShapeDtypeStruct((BP, DO), jnp.float32),
    )(x_p, s, w1, w2, b2)


RCO = 80
NCO = RPW // RCO


@functools.partial(
    pl.kernel,
    out_type=jax.ShapeDtypeStruct((BP, DO), jnp.float32),
    scratch_types=[
        pltpu.VMEM((NCO, RCO), jnp.int32),
        pltpu.VMEM((2, RCO, DO), jnp.float32),
        [pltpu.SemaphoreType.DMA] * 2,
    ],
    **_MESH,
)
def _gout(g_hbm, f2_hbm, o_hbm, idx_v, buf_v, sems):
    w = _wid()
    pltpu.sync_copy(f2_hbm.at[pl.ds(w * NCO, NCO)], idx_v)
    pltpu.async_copy(g_hbm.at[idx_v.at[0]], buf_v.at[0], sems[0])

    def outer(i, carry):
        c0 = i * 2
        for b in range(2):
            c = c0 + b

            @pl.when(c + 1 < NCO)
            def _():
                pltpu.async_copy(
                    g_hbm.at[idx_v.at[c + 1]], buf_v.at[1 - b], sems[1 - b]
                )

            pltpu.make_async_copy(
                g_hbm.at[idx_v.at[c]], buf_v.at[b], sems[b]
            ).wait()
            pltpu.sync_copy(buf_v.at[b], o_hbm.at[pl.ds(w * RPW + c * RCO, RCO)])
        return carry

    lax.fori_loop(0, NCO // 2, outer, 0)


def kernel(x, nodes, neighs, feats, W, b):
    nodes_p = jnp.concatenate(
        [nodes.astype(jnp.int32), jnp.zeros((BP - B,), jnp.int32)]
    )
    neighs_p = jnp.concatenate(
        [neighs.astype(jnp.int32), jnp.zeros((BP - B, K), jnp.int32)]
    )
    nf = jnp.reshape(neighs_p, (BP * K // 128, 128))
    x_p = jnp.concatenate([x, jnp.zeros((BP - B, F), jnp.float32)])
    w1 = W[:F]
    w2 = W[F:]
    b2 = jnp.reshape(b, (1, DO))

    f = _fmap(nodes_p)
    f2 = jnp.reshape(f, (BP // RCO, RCO))
    s = _gsum(nf, feats)
    g = _matmul(x_p, s, w1, w2, b2)
    out = _gout(g, f2)
    return out[:B]
